# baseline (speedup 1.0000x reference)
import functools

import jax
import jax.numpy as jnp
from jax import lax
from jax.experimental import pallas as pl
from jax.experimental.pallas import tpu as pltpu
from jax.experimental.pallas import tpu_sc as plsc

_B = 128
_D = 512
_N = _B * _D
_K = 2048
_NC = 2
_NS = 16
_L = 16
_NW = _NC * _NS
_EPW = _N // _NW
_ROWS = _EPW // _D


@functools.partial(
    pl.kernel,
    out_type=(
        jax.ShapeDtypeStruct((_B, _D), jnp.float32),
        jax.ShapeDtypeStruct((_NW, _L), jnp.float32),
    ),
    mesh=plsc.VectorSubcoreMesh(core_axis_name="c", subcore_axis_name="s",
                                num_cores=_NC, num_subcores=_NS),
    compiler_params=pltpu.CompilerParams(needs_layout_passes=False),
    scratch_types=[
        pltpu.VMEM((_K,), jnp.float32),
        pltpu.VMEM((_ROWS, _D), jnp.float32),
        pltpu.VMEM((_ROWS, _D), jnp.float32),
        pltpu.VMEM((_L,), jnp.float32),
    ],
)
def _vq_snap(x_hbm, emb_hbm, out_hbm, sq_hbm, emb_v, x_v, o_v, acc_v):
    cid = lax.axis_index("c")
    sid = lax.axis_index("s")
    wid = sid * _NC + cid
    lanes = lax.iota(jnp.int32, _L)

    pltpu.sync_copy(emb_hbm.at[0], emb_v)
    row0 = wid * _ROWS
    with jax.named_scope("x_dma"):
        pltpu.sync_copy(x_hbm.at[pl.ds(row0, _ROWS)], x_v)

    with jax.named_scope("minmax"):
        def mm(jv, carry):
            lo, hi = carry
            cjv = emb_v[pl.ds(jv * _L, _L)]
            return jnp.minimum(lo, cjv), jnp.maximum(hi, cjv)

        lo, hi = lax.fori_loop(
            0, _K // _L, mm,
            (jnp.full((_L,), jnp.inf, jnp.float32),
             jnp.full((_L,), -jnp.inf, jnp.float32)))
        smin = jnp.full((_L,), jnp.min(lo))
        smax = jnp.full((_L,), jnp.max(hi))

    def brute_lane(xv, l_splat, lc):
        xb = jnp.take_along_axis(xv, l_splat, axis=0)

        def scan_codes(jv, carry):
            dmin, val, idx = carry
            cjv = emb_v[pl.ds(jv * _L, _L)]
            d = (cjv - xb) * (cjv - xb)
            p = d < dmin
            jvec = jv * _L + lanes
            return (jnp.where(p, d, dmin), jnp.where(p, cjv, val),
                    jnp.where(p, jvec, idx))

        big = jnp.full((_L,), 3.4e38, jnp.float32)
        dmin, val, idx = lax.fori_loop(
            0, _K // _L, scan_codes,
            (big, jnp.zeros((_L,), jnp.float32),
             jnp.zeros((_L,), jnp.int32)))
        g = jnp.min(dmin)
        cand = jnp.where(dmin == g, idx, _K)
        bi = jnp.min(cand)
        value = jnp.max(jnp.where(cand == bi, val, -3.4e38))
        return jnp.where(lanes == l_splat, value, lc)

    def slow_path(xv, out_m, lc0):
        def cond(carry):
            in_m, _ = carry
            return plsc.all_reduce_population_count(in_m)[0] > 0

        def body(carry):
            in_m, lc = carry
            l_splat = plsc.all_reduce_ffs(in_m)
            lc = brute_lane(xv, l_splat, lc)
            return in_m & (lanes != l_splat), lc

        _, lc = lax.while_loop(cond, body, (~out_m, lc0))
        return lc

    with jax.named_scope("search_phase"):
        vregs_per_row = _D // _L
        _G = 4

        @plsc.parallel_loop(0, _EPW // _L // _G, unroll=4,
                            carry=jnp.zeros((_L,), jnp.float32))
        def acc(g, a):
            r = g // (vregs_per_row // _G)
            col0 = (g % (vregs_per_row // _G)) * (_G * _L)
            xs, his, outs = [], [], []
            all_out = None
            for u in range(_G):
                xv = x_v[r, pl.ds(col0 + u * _L, _L)]
                hi_m = xv >= smax
                out_m = (xv <= smin) | hi_m
                xs.append(xv)
                his.append(hi_m)
                outs.append(out_m)
                all_out = out_m if all_out is None else (all_out & out_m)
            n_out = plsc.all_reduce_population_count(all_out)

            def fast():
                return tuple(jnp.where(h, smax, smin) for h in his)

            def slow():
                lcs = []
                for u in range(_G):
                    lc0 = jnp.where(his[u], smax, smin)
                    n_u = plsc.all_reduce_population_count(outs[u])
                    lcs.append(lax.cond(
                        n_u[0] == _L,
                        lambda lc0=lc0: lc0,
                        lambda u=u, lc0=lc0: slow_path(xs[u], outs[u], lc0)))
                return tuple(lcs)

            lcs = lax.cond(n_out[0] == _L, fast, slow)
            for u in range(_G):
                xv = xs[u]
                lc = lcs[u]
                o_v[r, pl.ds(col0 + u * _L, _L)] = xv + (lc - xv)
                d = lc - xv
                a = a + d * d
            return a

    acc_v[...] = acc
    pltpu.sync_copy(o_v, out_hbm.at[pl.ds(row0, _ROWS)])
    pltpu.sync_copy(acc_v, sq_hbm.at[wid])


def kernel(encoded, embeddings):
    latent_code_st, sq = _vq_snap(encoded, embeddings)
    loss = 2.0 * (jnp.sum(sq) / encoded.shape[0])
    return latent_code_st, loss


# speedup vs baseline: 1.0422x; 1.0422x over previous
import functools

import jax
import jax.numpy as jnp
from jax import lax
from jax.experimental import pallas as pl
from jax.experimental.pallas import tpu as pltpu
from jax.experimental.pallas import tpu_sc as plsc

_B = 128
_D = 512
_N = _B * _D
_K = 2048
_NC = 2
_NS = 16
_L = 16
_NW = _NC * _NS
_EPW = _N // _NW
_ROWS = _EPW // _D


@functools.partial(
    pl.kernel,
    out_type=(
        jax.ShapeDtypeStruct((_B, _D), jnp.float32),
        jax.ShapeDtypeStruct((_NW, _L), jnp.float32),
    ),
    mesh=plsc.VectorSubcoreMesh(core_axis_name="c", subcore_axis_name="s",
                                num_cores=_NC, num_subcores=_NS),
    compiler_params=pltpu.CompilerParams(needs_layout_passes=False),
    scratch_types=[
        pltpu.VMEM((_K,), jnp.float32),
        pltpu.VMEM((_ROWS, _D), jnp.float32),
        pltpu.VMEM((_ROWS, _D), jnp.float32),
        pltpu.VMEM((_L,), jnp.float32),
    ],
)
def _vq_snap(x_hbm, emb_hbm, out_hbm, sq_hbm, emb_v, x_v, o_v, acc_v):
    cid = lax.axis_index("c")
    sid = lax.axis_index("s")
    wid = sid * _NC + cid
    lanes = lax.iota(jnp.int32, _L)

    pltpu.sync_copy(emb_hbm.at[0], emb_v)
    row0 = wid * _ROWS
    with jax.named_scope("x_dma"):
        pltpu.sync_copy(x_hbm.at[pl.ds(row0, _ROWS)], x_v)

    with jax.named_scope("minmax"):
        def mm(jv, carry):
            lo, hi = carry
            cjv = emb_v[pl.ds(jv * _L, _L)]
            return jnp.minimum(lo, cjv), jnp.maximum(hi, cjv)

        lo, hi = lax.fori_loop(
            0, _K // _L, mm,
            (jnp.full((_L,), jnp.inf, jnp.float32),
             jnp.full((_L,), -jnp.inf, jnp.float32)))
        smin = jnp.full((_L,), jnp.min(lo))
        smax = jnp.full((_L,), jnp.max(hi))

    def brute_lane(xv, l_splat, lc):
        xb = jnp.take_along_axis(xv, l_splat, axis=0)

        def scan_codes(jv, carry):
            dmin, val, idx = carry
            cjv = emb_v[pl.ds(jv * _L, _L)]
            d = (cjv - xb) * (cjv - xb)
            p = d < dmin
            jvec = jv * _L + lanes
            return (jnp.where(p, d, dmin), jnp.where(p, cjv, val),
                    jnp.where(p, jvec, idx))

        big = jnp.full((_L,), 3.4e38, jnp.float32)
        dmin, val, idx = lax.fori_loop(
            0, _K // _L, scan_codes,
            (big, jnp.zeros((_L,), jnp.float32),
             jnp.zeros((_L,), jnp.int32)))
        g = jnp.min(dmin)
        cand = jnp.where(dmin == g, idx, _K)
        bi = jnp.min(cand)
        value = jnp.max(jnp.where(cand == bi, val, -3.4e38))
        return jnp.where(lanes == l_splat, value, lc)

    def slow_path(xv, out_m, lc0):
        def cond(carry):
            in_m, _ = carry
            return plsc.all_reduce_population_count(in_m)[0] > 0

        def body(carry):
            in_m, lc = carry
            l_splat = plsc.all_reduce_ffs(in_m)
            lc = brute_lane(xv, l_splat, lc)
            return in_m & (lanes != l_splat), lc

        _, lc = lax.while_loop(cond, body, (~out_m, lc0))
        return lc

    with jax.named_scope("search_phase"):
        vregs_per_row = _D // _L
        _G = 4

        @plsc.parallel_loop(0, _EPW // _L // _G, unroll=2,
                            carry=jnp.zeros((_L,), jnp.float32))
        def acc(g, a):
            r = g // (vregs_per_row // _G)
            col0 = (g % (vregs_per_row // _G)) * (_G * _L)
            xs, his, outs = [], [], []
            all_out = None
            for u in range(_G):
                xv = x_v[r, pl.ds(col0 + u * _L, _L)]
                hi_m = xv >= smax
                out_m = (xv <= smin) | hi_m
                xs.append(xv)
                his.append(hi_m)
                outs.append(out_m)
                all_out = out_m if all_out is None else (all_out & out_m)
            n_out = plsc.all_reduce_population_count(all_out)

            def fast():
                return tuple(jnp.where(h, smax, smin) for h in his)

            def slow():
                lcs = []
                for u in range(_G):
                    lc0 = jnp.where(his[u], smax, smin)
                    n_u = plsc.all_reduce_population_count(outs[u])
                    lcs.append(lax.cond(
                        n_u[0] == _L,
                        lambda lc0=lc0: lc0,
                        lambda u=u, lc0=lc0: slow_path(xs[u], outs[u], lc0)))
                return tuple(lcs)

            lcs = lax.cond(n_out[0] == _L, fast, slow)
            for u in range(_G):
                xv = xs[u]
                lc = lcs[u]
                o_v[r, pl.ds(col0 + u * _L, _L)] = xv + (lc - xv)
                d = lc - xv
                a = a + d * d
            return a

    acc_v[...] = acc
    pltpu.sync_copy(o_v, out_hbm.at[pl.ds(row0, _ROWS)])
    pltpu.sync_copy(acc_v, sq_hbm.at[wid])


def kernel(encoded, embeddings):
    latent_code_st, sq = _vq_snap(encoded, embeddings)
    loss = 2.0 * (jnp.sum(sq) / encoded.shape[0])
    return latent_code_st, loss


# speedup vs baseline: 1.0617x; 1.0187x over previous
import functools

import jax
import jax.numpy as jnp
from jax import lax
from jax.experimental import pallas as pl
from jax.experimental.pallas import tpu as pltpu
from jax.experimental.pallas import tpu_sc as plsc

_B = 128
_D = 512
_N = _B * _D
_K = 2048
_NC = 2
_NS = 16
_L = 16
_NW = _NC * _NS
_EPW = _N // _NW
_ROWS = _EPW // _D


@functools.partial(
    pl.kernel,
    out_type=(
        jax.ShapeDtypeStruct((_B, _D), jnp.float32),
        jax.ShapeDtypeStruct((_NW, _L), jnp.float32),
    ),
    mesh=plsc.VectorSubcoreMesh(core_axis_name="c", subcore_axis_name="s",
                                num_cores=_NC, num_subcores=_NS),
    compiler_params=pltpu.CompilerParams(needs_layout_passes=False),
    scratch_types=[
        pltpu.VMEM((_K,), jnp.float32),
        pltpu.VMEM((_ROWS, _D), jnp.float32),
        pltpu.VMEM((_ROWS, _D), jnp.float32),
        pltpu.VMEM((_L,), jnp.float32),
        pltpu.SemaphoreType.DMA,
        pltpu.SemaphoreType.DMA,
    ],
)
def _vq_snap(x_hbm, emb_hbm, out_hbm, sq_hbm, emb_v, x_v, o_v, acc_v,
             emb_sem, x_sem):
    cid = lax.axis_index("c")
    sid = lax.axis_index("s")
    wid = sid * _NC + cid
    lanes = lax.iota(jnp.int32, _L)

    row0 = wid * _ROWS
    with jax.named_scope("x_dma"):
        emb_cp = pltpu.async_copy(emb_hbm.at[0], emb_v, emb_sem)
        x_cp = pltpu.async_copy(x_hbm.at[pl.ds(row0, _ROWS)], x_v, x_sem)
        emb_cp.wait()

    with jax.named_scope("minmax"):
        def mm(jv, carry):
            lo, hi = carry
            cjv = emb_v[pl.ds(jv * _L, _L)]
            return jnp.minimum(lo, cjv), jnp.maximum(hi, cjv)

        lo, hi = lax.fori_loop(
            0, _K // _L, mm,
            (jnp.full((_L,), jnp.inf, jnp.float32),
             jnp.full((_L,), -jnp.inf, jnp.float32)))
        smin = jnp.full((_L,), jnp.min(lo))
        smax = jnp.full((_L,), jnp.max(hi))
    x_cp.wait()

    def brute_lane(xv, l_splat, lc):
        xb = jnp.take_along_axis(xv, l_splat, axis=0)

        def scan_codes(jv, carry):
            dmin, val, idx = carry
            cjv = emb_v[pl.ds(jv * _L, _L)]
            d = (cjv - xb) * (cjv - xb)
            p = d < dmin
            jvec = jv * _L + lanes
            return (jnp.where(p, d, dmin), jnp.where(p, cjv, val),
                    jnp.where(p, jvec, idx))

        big = jnp.full((_L,), 3.4e38, jnp.float32)
        dmin, val, idx = lax.fori_loop(
            0, _K // _L, scan_codes,
            (big, jnp.zeros((_L,), jnp.float32),
             jnp.zeros((_L,), jnp.int32)))
        g = jnp.min(dmin)
        cand = jnp.where(dmin == g, idx, _K)
        bi = jnp.min(cand)
        value = jnp.max(jnp.where(cand == bi, val, -3.4e38))
        return jnp.where(lanes == l_splat, value, lc)

    def slow_path(xv, out_m, lc0):
        def cond(carry):
            in_m, _ = carry
            return plsc.all_reduce_population_count(in_m)[0] > 0

        def body(carry):
            in_m, lc = carry
            l_splat = plsc.all_reduce_ffs(in_m)
            lc = brute_lane(xv, l_splat, lc)
            return in_m & (lanes != l_splat), lc

        _, lc = lax.while_loop(cond, body, (~out_m, lc0))
        return lc

    with jax.named_scope("search_phase"):
        vregs_per_row = _D // _L
        _G = 4

        @plsc.parallel_loop(0, _EPW // _L // _G, unroll=2,
                            carry=jnp.zeros((_L,), jnp.float32))
        def acc(g, a):
            r = g // (vregs_per_row // _G)
            col0 = (g % (vregs_per_row // _G)) * (_G * _L)
            xs, his, outs = [], [], []
            all_out = None
            for u in range(_G):
                xv = x_v[r, pl.ds(col0 + u * _L, _L)]
                hi_m = xv >= smax
                out_m = (xv <= smin) | hi_m
                xs.append(xv)
                his.append(hi_m)
                outs.append(out_m)
                all_out = out_m if all_out is None else (all_out & out_m)
            n_out = plsc.all_reduce_population_count(all_out)

            def fast():
                return tuple(jnp.where(h, smax, smin) for h in his)

            def slow():
                lcs = []
                for u in range(_G):
                    lc0 = jnp.where(his[u], smax, smin)
                    n_u = plsc.all_reduce_population_count(outs[u])
                    lcs.append(lax.cond(
                        n_u[0] == _L,
                        lambda lc0=lc0: lc0,
                        lambda u=u, lc0=lc0: slow_path(xs[u], outs[u], lc0)))
                return tuple(lcs)

            lcs = lax.cond(n_out[0] == _L, fast, slow)
            for u in range(_G):
                xv = xs[u]
                lc = lcs[u]
                o_v[r, pl.ds(col0 + u * _L, _L)] = xv + (lc - xv)
                d = lc - xv
                a = a + d * d
            return a

    acc_v[...] = acc
    pltpu.sync_copy(o_v, out_hbm.at[pl.ds(row0, _ROWS)])
    pltpu.sync_copy(acc_v, sq_hbm.at[wid])


def kernel(encoded, embeddings):
    latent_code_st, sq = _vq_snap(encoded, embeddings)
    loss = 2.0 * (jnp.sum(sq) / encoded.shape[0])
    return latent_code_st, loss
